# baseline (speedup 1.0000x reference)
import functools

import jax
import jax.numpy as jnp
from jax import lax
from jax.experimental import pallas as pl
from jax.experimental.pallas import tpu as pltpu
from jax.experimental.pallas import tpu_sc as plsc

N = 10000
NPAD = 10240
E = 320000
CH = 128
NCHUNK = 2560
EPAD = NCHUNK * CH
NSUB = 16
NCORE = 2
CPW = NCHUNK // (NSUB * NCORE)
CPS1 = NCHUNK // NSUB
CPS2 = NCHUNK // (NSUB * NCORE)
G = 40
STRIPE = NPAD // NSUB

_mesh = plsc.VectorSubcoreMesh(core_axis_name="c", subcore_axis_name="s")


@functools.partial(
    pl.kernel,
    out_type=jax.ShapeDtypeStruct((NCORE, NPAD), jnp.float32),
    mesh=_mesh,
    scratch_types=[
        pltpu.VMEM((CPW, CH), jnp.int32),
        pltpu.VMEM((CH,), jnp.float32),
        pltpu.VMEM((STRIPE,), jnp.float32),
        pltpu.VMEM_SHARED((NPAD,), jnp.float32),
        pltpu.SemaphoreType.DMA,
    ],
)
def _deg_kernel(adj_hbm, deg_hbm, idx_v, ones_v, zeros_v, deg_sh, sem):
    c = lax.axis_index("c")
    s = lax.axis_index("s")
    w = c * NSUB + s

    @pl.loop(0, CH, step=16)
    def _(i):
        ones_v[pl.ds(i, 16)] = jnp.ones((16,), jnp.float32)

    @pl.loop(0, STRIPE, step=16)
    def _(i):
        zeros_v[pl.ds(i, 16)] = jnp.zeros((16,), jnp.float32)

    pltpu.sync_copy(zeros_v, deg_sh.at[pl.ds(s * STRIPE, STRIPE)])
    pltpu.sync_copy(adj_hbm.at[1, pl.ds(w * CPW, CPW)], idx_v)
    plsc.subcore_barrier()

    @pl.loop(0, CPW)
    def _(j):
        pltpu.make_async_copy(ones_v, deg_sh.at[idx_v.at[j]],
                              sem).start(add=True)

    @pl.loop(0, CPW)
    def _(j):
        pltpu.make_async_copy(ones_v, deg_sh.at[idx_v.at[j]], sem).wait()

    plsc.subcore_barrier()
    pltpu.sync_copy(deg_sh.at[pl.ds(s * STRIPE, STRIPE)],
                    deg_hbm.at[c, pl.ds(s * STRIPE, STRIPE)])


def _make_prop(per_core_table):
    cps = CPS1 if per_core_table else CPS2

    @functools.partial(
        pl.kernel,
        out_type=jax.ShapeDtypeStruct((NCORE, NPAD, 128), jnp.float32),
        mesh=_mesh,
        scratch_types=[
            pltpu.VMEM((G, CH), jnp.int32),
            pltpu.VMEM((G, CH), jnp.int32),
            pltpu.VMEM((CH, 128), jnp.float32),
            pltpu.VMEM((CH, 128), jnp.float32),
            pltpu.VMEM_SHARED((NPAD, 128), jnp.float32),
            pltpu.SemaphoreType.DMA,
            pltpu.SemaphoreType.DMA,
            pltpu.SemaphoreType.DMA,
            pltpu.SemaphoreType.DMA,
        ],
    )
    def _prop(table_hbm, adj_hbm, z_hbm, src_v, dst_v,
              rows0, rows1, z_sh, gsem0, gsem1, ssem0, ssem1):
        c = lax.axis_index("c")
        s = lax.axis_index("s")
        table = table_hbm.at[c] if per_core_table else table_hbm
        if per_core_table:
            base = s * cps
        else:
            base = c * (NCHUNK // NCORE) + s * cps

        def gather(j, rows, sem):
            return pltpu.make_async_copy(table.at[src_v.at[j]], rows, sem)

        def scatter(j, rows, sem):
            return pltpu.make_async_copy(rows, z_sh.at[dst_v.at[j]], sem)

        pltpu.sync_copy(table.at[pl.ds(s * STRIPE, STRIPE)],
                        z_sh.at[pl.ds(s * STRIPE, STRIPE)])
        plsc.subcore_barrier()

        @pl.loop(0, cps, step=G)
        def _(g):
            ls = pltpu.make_async_copy(
                adj_hbm.at[0, pl.ds(base + g, G)], src_v, gsem0)
            ld = pltpu.make_async_copy(
                adj_hbm.at[1, pl.ds(base + g, G)], dst_v, gsem1)
            ls.start()
            ld.start()
            ls.wait()
            ld.wait()
            gather(0, rows0, gsem0).start()

            @pl.loop(0, G, step=2)
            def _(j):
                @pl.when(j > 0)
                def _():
                    scatter(j, rows1, ssem1).wait()
                gather(j + 1, rows1, gsem1).start()
                gather(j, rows0, gsem0).wait()
                scatter(j, rows0, ssem0).start(add=True)
                scatter(j, rows0, ssem0).wait()

                @pl.when(j + 2 < G)
                def _():
                    gather(j + 2, rows0, gsem0).start()
                gather(j + 1, rows1, gsem1).wait()
                scatter(j + 1, rows1, ssem1).start(add=True)

            scatter(0, rows1, ssem1).wait()

        plsc.subcore_barrier()
        pltpu.sync_copy(z_sh.at[pl.ds(s * STRIPE, STRIPE)],
                        z_hbm.at[c, pl.ds(s * STRIPE, STRIPE)])

    return _prop


_prop1 = _make_prop(True)
_prop2 = _make_prop(False)


_BM = 10240


def _dot(a, b):
    return jax.lax.dot_general(a, b, (((1,), (0,)), ((), ())),
                               precision=lax.Precision.DEFAULT,
                               preferred_element_type=jnp.float32)


def _dinv_of(deg_ref):
    d = deg_ref[...]
    return lax.rsqrt(1.0 + d[0] + d[1])[:, None]


def _mm_scale(x, w, deg2):
    def body(x_ref, w_ref, deg_ref, o_ref, dinv_ref):
        dinv = _dinv_of(deg_ref)
        y = _dot(x_ref[...], w_ref[...])
        o_ref[0] = y[:, :128] * dinv
        o_ref[1] = y[:, 128:] * dinv
        dinv_ref[...] = dinv

    return pl.pallas_call(
        body,
        grid=(NPAD // _BM,),
        in_specs=[pl.BlockSpec((_BM, 128), lambda i: (i, 0)),
                  pl.BlockSpec((128, 256), lambda i: (0, 0)),
                  pl.BlockSpec((2, _BM), lambda i: (0, i))],
        out_specs=[pl.BlockSpec((2, _BM, 128), lambda i: (0, i, 0)),
                   pl.BlockSpec((_BM, 1), lambda i: (i, 0))],
        out_shape=[jax.ShapeDtypeStruct((2, NPAD, 128), jnp.float32),
                   jax.ShapeDtypeStruct((NPAD, 1), jnp.float32)],
    )(x, w, deg2)


def _layer2(z1, dinv1, b1, w2):
    def body(z_ref, dinv_ref, b1_ref, w2_ref, o_ref):
        dinv = dinv_ref[...]
        b = b1_ref[...]
        h0 = jnp.maximum(z_ref[0] * dinv + b[None, :128], 0.0)
        h1 = jnp.maximum(z_ref[1] * dinv + b[None, 128:], 0.0)
        y = _dot(h0, w2_ref[:128]) + _dot(h1, w2_ref[128:])
        o_ref[...] = y * dinv

    return pl.pallas_call(
        body,
        grid=(NPAD // _BM,),
        in_specs=[pl.BlockSpec((2, _BM, 128), lambda i: (0, i, 0)),
                  pl.BlockSpec((_BM, 1), lambda i: (i, 0)),
                  pl.BlockSpec((256,), lambda i: (0,)),
                  pl.BlockSpec((256, 128), lambda i: (0, 0))],
        out_specs=pl.BlockSpec((_BM, 128), lambda i: (i, 0)),
        out_shape=jax.ShapeDtypeStruct((NPAD, 128), jnp.float32),
    )(z1, dinv1, b1, w2)


_BMF = 10000


def _finalize(z2, xws2, dinv1, b2):
    def body(z_ref, t_ref, dinv_ref, b2_ref, o_ref):
        dinv = dinv_ref[...]
        b = b2_ref[...]
        o_ref[...] = (z_ref[0] + z_ref[1] - t_ref[...]) * dinv + b[None, :]

    return pl.pallas_call(
        body,
        grid=(N // _BMF,),
        in_specs=[pl.BlockSpec((2, _BMF, 128), lambda i: (0, i, 0)),
                  pl.BlockSpec((_BMF, 128), lambda i: (i, 0)),
                  pl.BlockSpec((_BMF, 1), lambda i: (i, 0)),
                  pl.BlockSpec((128,), lambda i: (0,))],
        out_specs=pl.BlockSpec((_BMF, 128), lambda i: (i, 0)),
        out_shape=jax.ShapeDtypeStruct((N, 128), jnp.float32),
    )(z2, xws2, dinv1, b2)


def kernel(x, adj_t, W1, b1, W2, b2):
    adjp = jnp.pad(adj_t.astype(jnp.int32),
                   ((0, 0), (0, EPAD - E))).reshape(2, NCHUNK, CH)
    chunkid = lax.broadcasted_iota(jnp.int32, (NCHUNK, CH), 0)
    col = lax.broadcasted_iota(jnp.int32, (NCHUNK, CH), 1)
    pos = chunkid * CH + col
    trash = N + pos % (NPAD - N)
    adj3 = jnp.where((pos < E)[None], adjp, trash[None])
    x_pad = jnp.pad(x, ((0, NPAD - N), (0, 0)))

    deg2 = _deg_kernel(adj3)
    xws1, dinv1 = _mm_scale(x_pad, W1, deg2)
    z1 = _prop1(xws1, adj3)
    xws2 = _layer2(z1, dinv1, b1, W2)
    z2 = _prop2(xws2, adj3)
    return _finalize(z2, xws2, dinv1, b2)


# speedup vs baseline: 1.0040x; 1.0040x over previous
import functools

import jax
import jax.numpy as jnp
from jax import lax
from jax.experimental import pallas as pl
from jax.experimental.pallas import tpu as pltpu
from jax.experimental.pallas import tpu_sc as plsc

N = 10000
NPAD = 10240
E = 320000
CH = 128
NCHUNK = 2560
EPAD = NCHUNK * CH
NSUB = 16
NCORE = 2
CPW = NCHUNK // (NSUB * NCORE)
CPS1 = NCHUNK // NSUB
CPS2 = NCHUNK // (NSUB * NCORE)
G = 40
STRIPE = NPAD // NSUB

_mesh = plsc.VectorSubcoreMesh(core_axis_name="c", subcore_axis_name="s")


@functools.partial(
    pl.kernel,
    out_type=jax.ShapeDtypeStruct((NCORE, NPAD), jnp.float32),
    mesh=_mesh,
    scratch_types=[
        pltpu.VMEM((CPW, CH), jnp.int32),
        pltpu.VMEM((CH,), jnp.float32),
        pltpu.VMEM((STRIPE,), jnp.float32),
        pltpu.VMEM_SHARED((NPAD,), jnp.float32),
        pltpu.SemaphoreType.DMA,
    ],
)
def _deg_kernel(adj_hbm, deg_hbm, idx_v, ones_v, zeros_v, deg_sh, sem):
    c = lax.axis_index("c")
    s = lax.axis_index("s")
    w = c * NSUB + s

    @pl.loop(0, CH, step=16)
    def _(i):
        ones_v[pl.ds(i, 16)] = jnp.ones((16,), jnp.float32)

    @pl.loop(0, STRIPE, step=16)
    def _(i):
        zeros_v[pl.ds(i, 16)] = jnp.zeros((16,), jnp.float32)

    pltpu.sync_copy(zeros_v, deg_sh.at[pl.ds(s * STRIPE, STRIPE)])
    pltpu.sync_copy(adj_hbm.at[1, pl.ds(w * CPW, CPW)], idx_v)
    plsc.subcore_barrier()

    @pl.loop(0, CPW)
    def _(j):
        pltpu.make_async_copy(ones_v, deg_sh.at[idx_v.at[j]],
                              sem).start(add=True)

    @pl.loop(0, CPW)
    def _(j):
        pltpu.make_async_copy(ones_v, deg_sh.at[idx_v.at[j]], sem).wait()

    plsc.subcore_barrier()
    pltpu.sync_copy(deg_sh.at[pl.ds(s * STRIPE, STRIPE)],
                    deg_hbm.at[c, pl.ds(s * STRIPE, STRIPE)])


def _make_prop(per_core_table):
    cps = CPS1 if per_core_table else CPS2

    @functools.partial(
        pl.kernel,
        out_type=jax.ShapeDtypeStruct((NCORE, NPAD, 128), jnp.float32),
        mesh=_mesh,
        scratch_types=[
            pltpu.VMEM((G, CH), jnp.int32),
            pltpu.VMEM((G, CH), jnp.int32),
            pltpu.VMEM((CH, 128), jnp.float32),
            pltpu.VMEM((CH, 128), jnp.float32),
            pltpu.VMEM_SHARED((NPAD, 128), jnp.float32),
            pltpu.SemaphoreType.DMA,
            pltpu.SemaphoreType.DMA,
            pltpu.SemaphoreType.DMA,
            pltpu.SemaphoreType.DMA,
        ],
    )
    def _prop(table_hbm, adj_hbm, z_hbm, src_v, dst_v,
              rows0, rows1, z_sh, gsem0, gsem1, ssem0, ssem1):
        c = lax.axis_index("c")
        s = lax.axis_index("s")
        table = table_hbm.at[c] if per_core_table else table_hbm
        if per_core_table:
            base = s * cps
        else:
            base = c * (NCHUNK // NCORE) + s * cps

        def gather(j, rows, sem):
            return pltpu.make_async_copy(table.at[src_v.at[j]], rows, sem)

        def scatter(j, rows, sem):
            return pltpu.make_async_copy(rows, z_sh.at[dst_v.at[j]], sem)

        pltpu.sync_copy(table.at[pl.ds(s * STRIPE, STRIPE)],
                        z_sh.at[pl.ds(s * STRIPE, STRIPE)])
        plsc.subcore_barrier()

        @pl.loop(0, cps, step=G)
        def _(g):
            ls = pltpu.make_async_copy(
                adj_hbm.at[0, pl.ds(base + g, G)], src_v, gsem0)
            ld = pltpu.make_async_copy(
                adj_hbm.at[1, pl.ds(base + g, G)], dst_v, gsem1)
            ls.start()
            ld.start()
            ls.wait()
            ld.wait()
            gather(0, rows0, gsem0).start()

            @pl.loop(0, G, step=2)
            def _(j):
                @pl.when(j > 0)
                def _():
                    scatter(j, rows1, ssem1).wait()
                gather(j + 1, rows1, gsem1).start()
                gather(j, rows0, gsem0).wait()
                scatter(j, rows0, ssem0).start(add=True)
                scatter(j, rows0, ssem0).wait()

                @pl.when(j + 2 < G)
                def _():
                    gather(j + 2, rows0, gsem0).start()
                gather(j + 1, rows1, gsem1).wait()
                scatter(j + 1, rows1, ssem1).start(add=True)

            scatter(0, rows1, ssem1).wait()

        plsc.subcore_barrier()
        pltpu.sync_copy(z_sh.at[pl.ds(s * STRIPE, STRIPE)],
                        z_hbm.at[c, pl.ds(s * STRIPE, STRIPE)])

    return _prop


_prop1 = _make_prop(True)
_prop2 = _make_prop(False)


_BM = 2560


def _dot(a, b):
    return jax.lax.dot_general(a, b, (((1,), (0,)), ((), ())),
                               precision=lax.Precision.DEFAULT,
                               preferred_element_type=jnp.float32)


def _dinv_of(deg_ref):
    d = deg_ref[...]
    return lax.rsqrt(1.0 + d[0] + d[1])[:, None]


def _mm_scale(x, w, deg2):
    def body(x_ref, w_ref, deg_ref, o_ref, dinv_ref):
        dinv = _dinv_of(deg_ref)
        y = _dot(x_ref[...], w_ref[...])
        o_ref[0] = y[:, :128] * dinv
        o_ref[1] = y[:, 128:] * dinv
        dinv_ref[...] = dinv

    return pl.pallas_call(
        body,
        grid=(NPAD // _BM,),
        in_specs=[pl.BlockSpec((_BM, 128), lambda i: (i, 0)),
                  pl.BlockSpec((128, 256), lambda i: (0, 0)),
                  pl.BlockSpec((2, _BM), lambda i: (0, i))],
        out_specs=[pl.BlockSpec((2, _BM, 128), lambda i: (0, i, 0)),
                   pl.BlockSpec((_BM, 1), lambda i: (i, 0))],
        out_shape=[jax.ShapeDtypeStruct((2, NPAD, 128), jnp.float32),
                   jax.ShapeDtypeStruct((NPAD, 1), jnp.float32)],
    )(x, w, deg2)


def _layer2(z1, dinv1, b1, w2):
    def body(z_ref, dinv_ref, b1_ref, w2_ref, o_ref):
        dinv = dinv_ref[...]
        b = b1_ref[...]
        h0 = jnp.maximum(z_ref[0] * dinv + b[None, :128], 0.0)
        h1 = jnp.maximum(z_ref[1] * dinv + b[None, 128:], 0.0)
        y = _dot(h0, w2_ref[:128]) + _dot(h1, w2_ref[128:])
        o_ref[...] = y * dinv

    return pl.pallas_call(
        body,
        grid=(NPAD // _BM,),
        in_specs=[pl.BlockSpec((2, _BM, 128), lambda i: (0, i, 0)),
                  pl.BlockSpec((_BM, 1), lambda i: (i, 0)),
                  pl.BlockSpec((256,), lambda i: (0,)),
                  pl.BlockSpec((256, 128), lambda i: (0, 0))],
        out_specs=pl.BlockSpec((_BM, 128), lambda i: (i, 0)),
        out_shape=jax.ShapeDtypeStruct((NPAD, 128), jnp.float32),
    )(z1, dinv1, b1, w2)


_BMF = 5000


def _finalize(z2, xws2, dinv1, b2):
    def body(z_ref, t_ref, dinv_ref, b2_ref, o_ref):
        dinv = dinv_ref[...]
        b = b2_ref[...]
        o_ref[...] = (z_ref[0] + z_ref[1] - t_ref[...]) * dinv + b[None, :]

    return pl.pallas_call(
        body,
        grid=(N // _BMF,),
        in_specs=[pl.BlockSpec((2, _BMF, 128), lambda i: (0, i, 0)),
                  pl.BlockSpec((_BMF, 128), lambda i: (i, 0)),
                  pl.BlockSpec((_BMF, 1), lambda i: (i, 0)),
                  pl.BlockSpec((128,), lambda i: (0,))],
        out_specs=pl.BlockSpec((_BMF, 128), lambda i: (i, 0)),
        out_shape=jax.ShapeDtypeStruct((N, 128), jnp.float32),
    )(z2, xws2, dinv1, b2)


def kernel(x, adj_t, W1, b1, W2, b2):
    adjp = jnp.pad(adj_t.astype(jnp.int32),
                   ((0, 0), (0, EPAD - E))).reshape(2, NCHUNK, CH)
    chunkid = lax.broadcasted_iota(jnp.int32, (NCHUNK, CH), 0)
    col = lax.broadcasted_iota(jnp.int32, (NCHUNK, CH), 1)
    pos = chunkid * CH + col
    trash = N + pos % (NPAD - N)
    adj3 = jnp.where((pos < E)[None], adjp, trash[None])
    x_pad = jnp.pad(x, ((0, NPAD - N), (0, 0)))

    deg2 = _deg_kernel(adj3)
    xws1, dinv1 = _mm_scale(x_pad, W1, deg2)
    z1 = _prop1(xws1, adj3)
    xws2 = _layer2(z1, dinv1, b1, W2)
    z2 = _prop2(xws2, adj3)
    return _finalize(z2, xws2, dinv1, b2)
